# baseline (speedup 1.0000x reference)
import numpy as np
import jax
import jax.numpy as jnp
from jax.experimental import pallas as pl
from jax.experimental.pallas import tpu as pltpu

IMG = 32
KS = 5
C_IN, C1, C2 = 3, 6, 16
H1 = IMG - KS + 1
P1H = H1 // 2
H2 = P1H - KS + 1
P2H = H2 // 2
FC1, FC2 = 120, 84
OUT_PAD = 128
PITCH = 128
KW = 5 * PITCH
N1 = C1 * IMG
N2C = C2 * 16


def _band(n):
    d = np.zeros((KS, n, n), np.float32)
    for k in range(KS):
        for w in range(n - k):
            d[k, w + k, w] = 1.0
    return d


_D1 = _band(IMG)
_D2 = _band(16)


def _selects():
    se1 = np.zeros((N1, PITCH), np.float32)
    so1 = np.zeros((N1, PITCH), np.float32)
    for c in range(C1):
        for wp in range(P1H):
            se1[c * IMG + 2 * wp, c * 16 + wp] = 1.0
            so1[c * IMG + 2 * wp + 1, c * 16 + wp] = 1.0
    se2 = np.zeros((N2C, PITCH), np.float32)
    so2 = np.zeros((N2C, PITCH), np.float32)
    for c in range(C2):
        for wp in range(P2H):
            se2[c * 16 + 2 * wp, c * 8 + wp] = 1.0
            so2[c * 16 + 2 * wp + 1, c * 8 + wp] = 1.0
    return se1, so1, se2, so2


_SE1, _SO1, _SE2, _SO2 = _selects()


def _lenet_body(x_ref, t1_ref, b1r_ref, t2_ref, b2r_ref,
                se1_ref, so1_ref, se2_ref, so2_ref,
                wf1_ref, f1b_ref, w2f_ref, f2b_ref, w3f_ref, f3b_ref,
                o_ref, p1_ref, f_ref):
    f32 = jnp.float32
    b1r = b1r_ref[...]
    se1 = se1_ref[...]
    so1 = so1_ref[...]

    bf16 = jnp.bfloat16

    def conv1_row(h):
        base = (h // 4) * PITCH
        acc = jnp.dot(x_ref[:, pl.ds(base, 256)], t1_ref[h % 4, 0],
                      preferred_element_type=f32)
        acc = acc + jnp.dot(x_ref[:, pl.ds(1024 + base, 256)],
                            t1_ref[h % 4, 1], preferred_element_type=f32)
        acc = acc + jnp.dot(x_ref[:, pl.ds(2048 + base, 256)],
                            t1_ref[h % 4, 2], preferred_element_type=f32)
        return jnp.maximum(acc + b1r, 0.0)

    for hp in range(P1H):
        pm = jnp.maximum(conv1_row(2 * hp), conv1_row(2 * hp + 1)).astype(bf16)
        p1_ref[:, pl.ds(hp * PITCH, PITCH)] = jnp.maximum(
            jnp.dot(pm, se1, preferred_element_type=f32),
            jnp.dot(pm, so1, preferred_element_type=f32)).astype(bf16)

    t2 = t2_ref[...]
    b2r = b2r_ref[...]
    se2 = se2_ref[...]
    so2 = so2_ref[...]
    for hp in range(P2H):
        r0 = jnp.maximum(
            jnp.dot(p1_ref[:, pl.ds(2 * hp * PITCH, KW)], t2,
                    preferred_element_type=f32) + b2r, 0.0)
        r1 = jnp.maximum(
            jnp.dot(p1_ref[:, pl.ds((2 * hp + 1) * PITCH, KW)],
                    t2, preferred_element_type=f32) + b2r, 0.0)
        pm = jnp.maximum(r0, r1).astype(bf16)
        f_ref[:, pl.ds(hp * PITCH, PITCH)] = jnp.maximum(
            jnp.dot(pm, se2, preferred_element_type=f32),
            jnp.dot(pm, so2, preferred_element_type=f32)).astype(bf16)
    h = jnp.maximum(
        jnp.dot(f_ref[...], wf1_ref[...], preferred_element_type=f32)
        + f1b_ref[...], 0.0)
    h = jnp.maximum(
        jnp.dot(h.astype(bf16), w2f_ref[...], preferred_element_type=f32)
        + f2b_ref[...], 0.0)
    o_ref[...] = (jnp.dot(h.astype(bf16), w3f_ref[...],
                          preferred_element_type=f32)
                  + f3b_ref[...])[:, :16]


def kernel(x, w1, b1, w2, b2, S2, fc1w, fc1b, fc2w, fc2b, fc3w, fc3b):
    del S2
    B = x.shape[0]
    f32 = jnp.float32
    bt = 512 if B >= 512 else max(8, B)
    n_blk = -(-B // bt)
    b_pad = n_blk * bt

    xr = x.astype(jnp.bfloat16).reshape(B, C_IN * IMG * IMG)
    if b_pad != B:
        xr = jnp.pad(xr, ((0, b_pad - B), (0, 0)))

    t1c = jnp.einsum('xdoc,djw->cxjow', w1.reshape(KS, KS, C1, C_IN),
                     jnp.asarray(_D1)).reshape(C_IN, KS * IMG, N1)
    t1 = jnp.stack([jnp.pad(t1c, ((0, 0), (p * IMG, 96 - p * IMG), (0, 0)))
                    for p in range(4)]).astype(jnp.bfloat16)
    t2 = jnp.einsum('xdoc,djw->xcjow', w2.reshape(KS, KS, C2, C1),
                    jnp.asarray(_D2)).reshape(KS, C1 * 16, N2C)
    t2 = jnp.pad(t2, ((0, 0), (0, 32), (0, 0))).reshape(KW, N2C)
    t2 = t2.astype(jnp.bfloat16)
    wf1 = jnp.pad(
        fc1w.reshape(C2, P2H, P2H, FC1).transpose(1, 0, 2, 3),
        ((0, 0), (0, 0), (0, 3), (0, 0))).reshape(P2H * PITCH, FC1)
    wf1 = wf1.astype(jnp.bfloat16)
    b1r = jnp.broadcast_to(b1, (C1, IMG)).reshape(1, N1)
    b2r = jnp.broadcast_to(b2, (C2, 16)).reshape(1, N2C)

    out = pl.pallas_call(
        _lenet_body,
        out_shape=jax.ShapeDtypeStruct((b_pad, 16), f32),
        grid=(n_blk,),
        in_specs=[
            pl.BlockSpec((bt, C_IN * IMG * IMG), lambda i: (i, 0)),
            pl.BlockSpec((4, C_IN, 256, N1), lambda i: (0, 0, 0, 0)),
            pl.BlockSpec((1, N1), lambda i: (0, 0)),
            pl.BlockSpec((KW, N2C), lambda i: (0, 0)),
            pl.BlockSpec((1, N2C), lambda i: (0, 0)),
            pl.BlockSpec((N1, PITCH), lambda i: (0, 0)),
            pl.BlockSpec((N1, PITCH), lambda i: (0, 0)),
            pl.BlockSpec((N2C, PITCH), lambda i: (0, 0)),
            pl.BlockSpec((N2C, PITCH), lambda i: (0, 0)),
            pl.BlockSpec((P2H * PITCH, FC1), lambda i: (0, 0)),
            pl.BlockSpec((1, FC1), lambda i: (0, 0)),
            pl.BlockSpec((FC1, FC2), lambda i: (0, 0)),
            pl.BlockSpec((1, FC2), lambda i: (0, 0)),
            pl.BlockSpec((FC2, OUT_PAD), lambda i: (0, 0)),
            pl.BlockSpec((1, OUT_PAD), lambda i: (0, 0)),
        ],
        out_specs=pl.BlockSpec((bt, 16), lambda i: (i, 0)),
        scratch_shapes=[
            pltpu.VMEM((bt, P1H * PITCH), jnp.bfloat16),
            pltpu.VMEM((bt, P2H * PITCH), jnp.bfloat16),
        ],
        compiler_params=pltpu.CompilerParams(
            dimension_semantics=("parallel",)),
    )(xr, t1, b1r, t2, b2r,
      jnp.asarray(_SE1, dtype=jnp.bfloat16),
      jnp.asarray(_SO1, dtype=jnp.bfloat16),
      jnp.asarray(_SE2, dtype=jnp.bfloat16),
      jnp.asarray(_SO2, dtype=jnp.bfloat16),
      wf1, fc1b, fc2w.astype(jnp.bfloat16), fc2b,
      fc3w.astype(jnp.bfloat16), fc3b)
    return out[:B, :10]


# speedup vs baseline: 1.0682x; 1.0682x over previous
import numpy as np
import jax
import jax.numpy as jnp
from jax.experimental import pallas as pl
from jax.experimental.pallas import tpu as pltpu

IMG = 32
KS = 5
C_IN, C1, C2 = 3, 6, 16
H1 = IMG - KS + 1
P1H = H1 // 2
H2 = P1H - KS + 1
P2H = H2 // 2
FC1, FC2 = 120, 84
OUT_PAD = 128
PITCH = 128
KW = 5 * PITCH
N1 = C1 * IMG
N2C = C2 * 16


def _band(n):
    d = np.zeros((KS, n, n), np.float32)
    for k in range(KS):
        for w in range(n - k):
            d[k, w + k, w] = 1.0
    return d


_D1 = _band(IMG)
_D2 = _band(16)


def _selects():
    se1 = np.zeros((N1, PITCH), np.float32)
    so1 = np.zeros((N1, PITCH), np.float32)
    for c in range(C1):
        for wp in range(P1H):
            se1[c * IMG + 2 * wp, c * 16 + wp] = 1.0
            so1[c * IMG + 2 * wp + 1, c * 16 + wp] = 1.0
    se2 = np.zeros((N2C, PITCH), np.float32)
    so2 = np.zeros((N2C, PITCH), np.float32)
    for c in range(C2):
        for wp in range(P2H):
            se2[c * 16 + 2 * wp, c * 8 + wp] = 1.0
            so2[c * 16 + 2 * wp + 1, c * 8 + wp] = 1.0
    return se1, so1, se2, so2


_SE1, _SO1, _SE2, _SO2 = _selects()


def _lenet_body(x_ref, t1_ref, b1r_ref, t2_ref, b2r_ref,
                se1_ref, so1_ref, se2_ref, so2_ref,
                wf1_ref, f1b_ref, w2f_ref, f2b_ref, w3f_ref, f3b_ref,
                o_ref, p1_ref, f_ref):
    f32 = jnp.float32
    b1r = b1r_ref[...]
    se1 = se1_ref[...]
    so1 = so1_ref[...]

    bf16 = jnp.bfloat16

    def conv1_row(h):
        base = (h // 4) * PITCH
        acc = jnp.dot(x_ref[:, pl.ds(base, 256)].astype(bf16),
                      t1_ref[h % 4, 0], preferred_element_type=f32)
        acc = acc + jnp.dot(x_ref[:, pl.ds(1024 + base, 256)].astype(bf16),
                            t1_ref[h % 4, 1], preferred_element_type=f32)
        acc = acc + jnp.dot(x_ref[:, pl.ds(2048 + base, 256)].astype(bf16),
                            t1_ref[h % 4, 2], preferred_element_type=f32)
        return jnp.maximum(acc + b1r, 0.0)

    for hp in range(P1H):
        pm = jnp.maximum(conv1_row(2 * hp), conv1_row(2 * hp + 1))
        p1_ref[:, pl.ds(hp * PITCH, PITCH)] = jnp.maximum(
            jnp.dot(pm, se1, preferred_element_type=f32),
            jnp.dot(pm, so1, preferred_element_type=f32))

    t2 = t2_ref[...]
    b2r = b2r_ref[...]
    se2 = se2_ref[...]
    so2 = so2_ref[...]
    for hp in range(P2H):
        r0 = jnp.maximum(
            jnp.dot(p1_ref[:, pl.ds(2 * hp * PITCH, KW)].astype(bf16), t2,
                    preferred_element_type=f32) + b2r, 0.0)
        r1 = jnp.maximum(
            jnp.dot(p1_ref[:, pl.ds((2 * hp + 1) * PITCH, KW)].astype(bf16),
                    t2, preferred_element_type=f32) + b2r, 0.0)
        pm = jnp.maximum(r0, r1)
        f_ref[:, pl.ds(hp * PITCH, PITCH)] = jnp.maximum(
            jnp.dot(pm, se2, preferred_element_type=f32),
            jnp.dot(pm, so2, preferred_element_type=f32))
    h = jnp.maximum(
        jnp.dot(f_ref[...], wf1_ref[...], preferred_element_type=f32)
        + f1b_ref[...], 0.0)
    h = jnp.maximum(
        jnp.dot(h, w2f_ref[...], preferred_element_type=f32)
        + f2b_ref[...], 0.0)
    o_ref[...] = (jnp.dot(h, w3f_ref[...], preferred_element_type=f32)
                  + f3b_ref[...])


def kernel(x, w1, b1, w2, b2, S2, fc1w, fc1b, fc2w, fc2b, fc3w, fc3b):
    del S2
    B = x.shape[0]
    f32 = jnp.float32
    bt = 512 if B >= 512 else max(8, B)
    n_blk = -(-B // bt)
    b_pad = n_blk * bt

    xr = x.astype(f32).reshape(B, C_IN * IMG * IMG)
    if b_pad != B:
        xr = jnp.pad(xr, ((0, b_pad - B), (0, 0)))

    t1c = jnp.einsum('xdoc,djw->cxjow', w1.reshape(KS, KS, C1, C_IN),
                     jnp.asarray(_D1)).reshape(C_IN, KS * IMG, N1)
    t1 = jnp.stack([jnp.pad(t1c, ((0, 0), (p * IMG, 96 - p * IMG), (0, 0)))
                    for p in range(4)]).astype(jnp.bfloat16)
    t2 = jnp.einsum('xdoc,djw->xcjow', w2.reshape(KS, KS, C2, C1),
                    jnp.asarray(_D2)).reshape(KS, C1 * 16, N2C)
    t2 = jnp.pad(t2, ((0, 0), (0, 32), (0, 0))).reshape(KW, N2C)
    t2 = t2.astype(jnp.bfloat16)
    wf1 = jnp.pad(
        fc1w.reshape(C2, P2H, P2H, FC1).transpose(1, 0, 2, 3),
        ((0, 0), (0, 0), (0, 3), (0, 0))).reshape(P2H * PITCH, FC1)
    b1r = jnp.broadcast_to(b1, (C1, IMG)).reshape(1, N1)
    b2r = jnp.broadcast_to(b2, (C2, 16)).reshape(1, N2C)

    out = pl.pallas_call(
        _lenet_body,
        out_shape=jax.ShapeDtypeStruct((b_pad, OUT_PAD), f32),
        grid=(n_blk,),
        in_specs=[
            pl.BlockSpec((bt, C_IN * IMG * IMG), lambda i: (i, 0)),
            pl.BlockSpec((4, C_IN, 256, N1), lambda i: (0, 0, 0, 0)),
            pl.BlockSpec((1, N1), lambda i: (0, 0)),
            pl.BlockSpec((KW, N2C), lambda i: (0, 0)),
            pl.BlockSpec((1, N2C), lambda i: (0, 0)),
            pl.BlockSpec((N1, PITCH), lambda i: (0, 0)),
            pl.BlockSpec((N1, PITCH), lambda i: (0, 0)),
            pl.BlockSpec((N2C, PITCH), lambda i: (0, 0)),
            pl.BlockSpec((N2C, PITCH), lambda i: (0, 0)),
            pl.BlockSpec((P2H * PITCH, FC1), lambda i: (0, 0)),
            pl.BlockSpec((1, FC1), lambda i: (0, 0)),
            pl.BlockSpec((FC1, FC2), lambda i: (0, 0)),
            pl.BlockSpec((1, FC2), lambda i: (0, 0)),
            pl.BlockSpec((FC2, OUT_PAD), lambda i: (0, 0)),
            pl.BlockSpec((1, OUT_PAD), lambda i: (0, 0)),
        ],
        out_specs=pl.BlockSpec((bt, OUT_PAD), lambda i: (i, 0)),
        scratch_shapes=[
            pltpu.VMEM((bt, P1H * PITCH), f32),
            pltpu.VMEM((bt, P2H * PITCH), f32),
        ],
        compiler_params=pltpu.CompilerParams(
            dimension_semantics=("parallel",),
            allow_input_fusion=[False] + [True] * 14),
    )(xr, t1, b1r, t2, b2r,
      jnp.asarray(_SE1), jnp.asarray(_SO1), jnp.asarray(_SE2),
      jnp.asarray(_SO2), wf1, fc1b, fc2w, fc2b, fc3w, fc3b)
    return out[:B, :10]
